# baseline (speedup 1.0000x reference)
import functools

import jax
import jax.numpy as jnp
from jax import lax
from jax.experimental import pallas as pl
from jax.experimental.pallas import tpu as pltpu
from jax.experimental.pallas import tpu_sc as plsc

N_NODES = 10000
N_EDGES = 320000
IN_RAW = 7
HIDDEN = 128
CODEBOOK = 1024

NC, NS = 2, 16
NW = NC * NS

NPAD = 10240
NZERO = NPAD - N_NODES

ENDP = 2 * N_EDGES
ENDP_PER_TILE = ENDP // NW
HW = 2000
H_WINS = ENDP_PER_TILE // HW

PAIRS = 2 * N_EDGES
SW = 128
S_WINS = 158
PAIRS_PER_TILE = SW * S_WINS
PAIRS_PAD = PAIRS_PER_TILE * NW

ROWS_D = 400
GRID_D = N_NODES // ROWS_D


@functools.cache
def _sc_mesh():
    return plsc.VectorSubcoreMesh(
        core_axis_name="c", subcore_axis_name="s",
        num_cores=NC, num_subcores=NS)


def _hist_body(endp_hbm, ones_hbm, zeros_hbm, hist_out, idx_v, ones_v,
               deg_sh, sem):
    c = lax.axis_index("c")
    s = lax.axis_index("s")
    wid = c * NS + s

    @pl.when(s == 0)
    def _():
        pltpu.sync_copy(zeros_hbm, deg_sh)
    pltpu.sync_copy(ones_hbm, ones_v)
    plsc.subcore_barrier()

    def win(w, carry):
        base = wid * ENDP_PER_TILE + w * HW
        pltpu.sync_copy(endp_hbm.at[pl.ds(base, HW)], idx_v)
        pltpu.sync_copy(ones_v, deg_sh.at[idx_v], add=True)
        return carry

    lax.fori_loop(0, H_WINS, win, 0)
    plsc.subcore_barrier()

    @pl.when(s == 0)
    def _():
        pltpu.sync_copy(deg_sh, hist_out.at[c])


@functools.cache
def _hist_call():
    return pl.kernel(
        _hist_body,
        out_type=jax.ShapeDtypeStruct((NC, NPAD), jnp.float32),
        mesh=_sc_mesh(),
        scratch_types=[
            pltpu.VMEM((HW,), jnp.int32),
            pltpu.VMEM((HW,), jnp.float32),
            pltpu.VMEM_SHARED((NPAD,), jnp.float32),
            pltpu.SemaphoreType.DMA,
        ],
    )


def _scat_body(srcs_hbm, dsts_hbm, hn_hbm, zeros_hbm, out_hbm,
               sidx0, didx0, sidx1, didx1, rows0, rows1, acc_sh,
               sem0, sem1):
    c = lax.axis_index("c")
    s = lax.axis_index("s")
    wid = c * NS + s
    base = wid * PAIRS_PER_TILE

    @pl.when(s == 0)
    def _():
        pltpu.sync_copy(zeros_hbm, acc_sh)
    plsc.subcore_barrier()

    pltpu.sync_copy(srcs_hbm.at[pl.ds(base, SW)], sidx0)
    pltpu.sync_copy(dsts_hbm.at[pl.ds(base, SW)], didx0)
    pltpu.async_copy(hn_hbm.at[sidx0], rows0, sem0)

    def body(i, carry):
        b1 = base + (2 * i + 1) * SW
        pltpu.sync_copy(srcs_hbm.at[pl.ds(b1, SW)], sidx1)
        pltpu.sync_copy(dsts_hbm.at[pl.ds(b1, SW)], didx1)
        pltpu.async_copy(hn_hbm.at[sidx1], rows1, sem1)

        pltpu.make_async_copy(hn_hbm.at[sidx0], rows0, sem0).wait()
        pltpu.sync_copy(rows0, acc_sh.at[didx0], add=True)

        @pl.when(i < S_WINS // 2 - 1)
        def _():
            b2 = base + (2 * i + 2) * SW
            pltpu.sync_copy(srcs_hbm.at[pl.ds(b2, SW)], sidx0)
            pltpu.sync_copy(dsts_hbm.at[pl.ds(b2, SW)], didx0)
            pltpu.async_copy(hn_hbm.at[sidx0], rows0, sem0)

        pltpu.make_async_copy(hn_hbm.at[sidx1], rows1, sem1).wait()
        pltpu.sync_copy(rows1, acc_sh.at[didx1], add=True)
        return carry

    lax.fori_loop(0, S_WINS // 2, body, 0)
    plsc.subcore_barrier()

    @pl.when(s == 0)
    def _():
        pltpu.sync_copy(acc_sh, out_hbm.at[c])


@functools.cache
def _scat_call():
    return pl.kernel(
        _scat_body,
        out_type=jax.ShapeDtypeStruct((NC, NPAD, HIDDEN), jnp.float32),
        mesh=_sc_mesh(),
        compiler_params=pltpu.CompilerParams(use_tc_tiling_on_sc=False),
        scratch_types=[
            pltpu.VMEM((SW,), jnp.int32),
            pltpu.VMEM((SW,), jnp.int32),
            pltpu.VMEM((SW,), jnp.int32),
            pltpu.VMEM((SW,), jnp.int32),
            pltpu.VMEM((SW, HIDDEN), jnp.float32),
            pltpu.VMEM((SW, HIDDEN), jnp.float32),
            pltpu.VMEM_SHARED((NPAD, HIDDEN), jnp.float32),
            pltpu.SemaphoreType.DMA,
            pltpu.SemaphoreType.DMA,
        ],
    )


def _prep_body(histT_ref, feats_ref, w2_ref, b2_ref, hn_ref, norm_ref):
    deg = histT_ref[:, 0:1] + histT_ref[:, 1:2] + 1.0
    nrm = lax.rsqrt(jnp.maximum(deg, 1.0))
    row = lax.broadcasted_iota(jnp.int32, (NPAD, 1), 0)
    nrm = jnp.where(row < N_NODES, nrm, 0.0)
    h0 = jnp.dot(feats_ref[...], w2_ref[...],
                 preferred_element_type=jnp.float32) + b2_ref[...]
    hn_ref[...] = h0 * nrm
    norm_ref[...] = nrm


def _prep_call(histT, feats_pad, W2, b2r):
    return pl.pallas_call(
        _prep_body,
        out_shape=[
            jax.ShapeDtypeStruct((NPAD, HIDDEN), jnp.float32),
            jax.ShapeDtypeStruct((NPAD, 1), jnp.float32),
        ],
    )(histT, feats_pad, W2, b2r)


def _main_body(scat0_ref, scat1_ref, hn_ref, norm_ref, wg_ref, bg_ref,
               ct_ref, cb_ref, csq_ref,
               h_ref, q_ref, d2_ref, loss_ref, ind_ref, acc_sm):
    i = pl.program_id(0)
    agg = (scat0_ref[0] + scat1_ref[0] + hn_ref[...]) * norm_ref[...]
    h = jnp.maximum(
        jnp.dot(agg, wg_ref[...], preferred_element_type=jnp.float32)
        + bg_ref[...], 0.0)
    hh = jnp.sum(h * h, axis=1, keepdims=True)
    hc = jnp.dot(h, ct_ref[...], preferred_element_type=jnp.float32)
    d2 = hh - 2.0 * hc + csq_ref[...]
    iota = lax.broadcasted_iota(jnp.int32, (ROWS_D, CODEBOOK), 1)
    mind = jnp.min(d2, axis=1, keepdims=True)
    ind2 = jnp.min(jnp.where(d2 == mind, iota, CODEBOOK),
                   axis=1, keepdims=True)
    oh = (iota == ind2).astype(jnp.float32)
    q = jnp.dot(oh, cb_ref[...], preferred_element_type=jnp.float32,
                precision=lax.Precision.HIGHEST)
    h_ref[...] = h
    q_ref[...] = q
    d2_ref[...] = d2
    ind_ref[...] = ind2

    diff = q - h
    part = jnp.sum(diff * diff)

    @pl.when(i == 0)
    def _():
        acc_sm[0] = 0.0
    acc_sm[0] += part
    loss_ref[...] = jnp.full((1, 1), 1.25 / (N_NODES * HIDDEN),
                             jnp.float32) * acc_sm[0]


def _main_call(scat, hn, nrm, wg, bg2, ct, cb, csq):
    return pl.pallas_call(
        _main_body,
        grid=(GRID_D,),
        in_specs=[
            pl.BlockSpec((1, ROWS_D, HIDDEN), lambda i: (0, i, 0)),
            pl.BlockSpec((1, ROWS_D, HIDDEN), lambda i: (1, i, 0)),
            pl.BlockSpec((ROWS_D, HIDDEN), lambda i: (i, 0)),
            pl.BlockSpec((ROWS_D, 1), lambda i: (i, 0)),
            pl.BlockSpec((HIDDEN, HIDDEN), lambda i: (0, 0)),
            pl.BlockSpec((1, HIDDEN), lambda i: (0, 0)),
            pl.BlockSpec((HIDDEN, CODEBOOK), lambda i: (0, 0)),
            pl.BlockSpec((CODEBOOK, HIDDEN), lambda i: (0, 0)),
            pl.BlockSpec((1, CODEBOOK), lambda i: (0, 0)),
        ],
        out_specs=[
            pl.BlockSpec((ROWS_D, HIDDEN), lambda i: (i, 0)),
            pl.BlockSpec((ROWS_D, HIDDEN), lambda i: (i, 0)),
            pl.BlockSpec((ROWS_D, CODEBOOK), lambda i: (i, 0)),
            pl.BlockSpec((1, 1), lambda i: (0, 0)),
            pl.BlockSpec((ROWS_D, 1), lambda i: (i, 0)),
        ],
        out_shape=[
            jax.ShapeDtypeStruct((N_NODES, HIDDEN), jnp.float32),
            jax.ShapeDtypeStruct((N_NODES, HIDDEN), jnp.float32),
            jax.ShapeDtypeStruct((N_NODES, CODEBOOK), jnp.float32),
            jax.ShapeDtypeStruct((1, 1), jnp.float32),
            jax.ShapeDtypeStruct((N_NODES, 1), jnp.int32),
        ],
        scratch_shapes=[pltpu.SMEM((1,), jnp.float32)],
    )(scat, scat, hn, nrm, wg, bg2, ct, cb, csq)


def kernel(feats, edge_index, epoch, W2, b2, Wg, bg, codebook):
    del epoch
    ei = edge_index.astype(jnp.int32)
    src, dst = ei[0], ei[1]

    endp = ei.reshape(-1)
    ones_w = jnp.ones((HW,), jnp.float32)
    zeros_n = jnp.zeros((NPAD,), jnp.float32)
    hist = _hist_call()(endp, ones_w, zeros_n)

    feats_pad = jnp.pad(feats, ((0, NPAD - N_NODES), (0, 0)))
    hn, nrm = _prep_call(hist.T, feats_pad, W2, b2[None, :])

    half_pad = (PAIRS_PAD - PAIRS) // 2
    ar = jnp.arange(half_pad, dtype=jnp.int32)
    pad_src = N_NODES + (ar % NZERO)
    pad_dst = ar % N_NODES
    srcs_all = jnp.concatenate([src, pad_src, dst, pad_src])
    dsts_all = jnp.concatenate([dst, pad_dst, src, pad_dst])
    zeros_nh = jnp.zeros((NPAD, HIDDEN), jnp.float32)
    scat = _scat_call()(srcs_all, dsts_all, hn, zeros_nh)

    bg2 = bg[None, :]
    ct = codebook.T
    csq = jnp.sum(codebook * codebook, axis=1)[None, :]

    h, q, d2, loss, ind = _main_call(
        scat, hn[:N_NODES], nrm[:N_NODES], Wg, bg2, ct, codebook, csq)
    return h, q, d2, loss[0, 0], ind[:, 0]


# speedup vs baseline: 1.0145x; 1.0145x over previous
import functools

import jax
import jax.numpy as jnp
from jax import lax
from jax.experimental import pallas as pl
from jax.experimental.pallas import tpu as pltpu
from jax.experimental.pallas import tpu_sc as plsc

N_NODES = 10000
N_EDGES = 320000
IN_RAW = 7
HIDDEN = 128
CODEBOOK = 1024

NC, NS = 2, 16
NW = NC * NS

NPAD = 10240
NZERO = NPAD - N_NODES

ENDP = 2 * N_EDGES
ENDP_PER_TILE = ENDP // NW
HW = 2000
H_WINS = ENDP_PER_TILE // HW

PAIRS = 2 * N_EDGES
SW = 512
S_WINS = 40
PAIRS_PER_TILE = SW * S_WINS
PAIRS_PAD = PAIRS_PER_TILE * NW
HHALF = HIDDEN // 2

ROWS_D = 400
GRID_D = N_NODES // ROWS_D


@functools.cache
def _sc_mesh():
    return plsc.VectorSubcoreMesh(
        core_axis_name="c", subcore_axis_name="s",
        num_cores=NC, num_subcores=NS)


def _hist_body(endp_hbm, ones_hbm, zeros_hbm, hist_out, idx_v, ones_v,
               deg_sh, sem):
    c = lax.axis_index("c")
    s = lax.axis_index("s")
    wid = c * NS + s

    @pl.when(s == 0)
    def _():
        pltpu.sync_copy(zeros_hbm, deg_sh)
    pltpu.sync_copy(ones_hbm, ones_v)
    plsc.subcore_barrier()

    def win(w, carry):
        base = wid * ENDP_PER_TILE + w * HW
        pltpu.sync_copy(endp_hbm.at[pl.ds(base, HW)], idx_v)
        pltpu.sync_copy(ones_v, deg_sh.at[idx_v], add=True)
        return carry

    lax.fori_loop(0, H_WINS, win, 0)
    plsc.subcore_barrier()

    @pl.when(s == 0)
    def _():
        pltpu.sync_copy(deg_sh, hist_out.at[c])


@functools.cache
def _hist_call():
    return pl.kernel(
        _hist_body,
        out_type=jax.ShapeDtypeStruct((NC, NPAD), jnp.float32),
        mesh=_sc_mesh(),
        scratch_types=[
            pltpu.VMEM((HW,), jnp.int32),
            pltpu.VMEM((HW,), jnp.float32),
            pltpu.VMEM_SHARED((NPAD,), jnp.float32),
            pltpu.SemaphoreType.DMA,
        ],
    )


def _scat_body(srcs_hbm, dsts_hbm, hn_hbm, zeros_hbm, out_hbm,
               sidx0, didx0, sidx1, didx1, rows0, rows1, acc_sh,
               sem0, sem1):
    c = lax.axis_index("c")
    s = lax.axis_index("s")
    wid = c * NS + s
    base = wid * PAIRS_PER_TILE

    @pl.when(s == 0)
    def _():
        pltpu.sync_copy(zeros_hbm, acc_sh)
    plsc.subcore_barrier()

    pltpu.sync_copy(srcs_hbm.at[pl.ds(base, SW)], sidx0)
    pltpu.sync_copy(dsts_hbm.at[pl.ds(base, SW)], didx0)
    pltpu.async_copy(hn_hbm.at[sidx0], rows0, sem0)

    def body(i, carry):
        b1 = base + (2 * i + 1) * SW
        pltpu.sync_copy(srcs_hbm.at[pl.ds(b1, SW)], sidx1)
        pltpu.sync_copy(dsts_hbm.at[pl.ds(b1, SW)], didx1)
        pltpu.async_copy(hn_hbm.at[sidx1], rows1, sem1)

        pltpu.make_async_copy(hn_hbm.at[sidx0], rows0, sem0).wait()
        pltpu.sync_copy(rows0, acc_sh.at[didx0], add=True)

        @pl.when(i < S_WINS // 2 - 1)
        def _():
            b2 = base + (2 * i + 2) * SW
            pltpu.sync_copy(srcs_hbm.at[pl.ds(b2, SW)], sidx0)
            pltpu.sync_copy(dsts_hbm.at[pl.ds(b2, SW)], didx0)
            pltpu.async_copy(hn_hbm.at[sidx0], rows0, sem0)

        pltpu.make_async_copy(hn_hbm.at[sidx1], rows1, sem1).wait()
        pltpu.sync_copy(rows1, acc_sh.at[didx1], add=True)
        return carry

    lax.fori_loop(0, S_WINS // 2, body, 0)
    plsc.subcore_barrier()

    @pl.when(s == 0)
    def _():
        pltpu.sync_copy(acc_sh, out_hbm.at[c])


@functools.cache
def _scat_call():
    return pl.kernel(
        _scat_body,
        out_type=jax.ShapeDtypeStruct((NC, NPAD, HHALF), jnp.float32),
        mesh=_sc_mesh(),
        compiler_params=pltpu.CompilerParams(use_tc_tiling_on_sc=False),
        scratch_types=[
            pltpu.VMEM((SW,), jnp.int32),
            pltpu.VMEM((SW,), jnp.int32),
            pltpu.VMEM((SW,), jnp.int32),
            pltpu.VMEM((SW,), jnp.int32),
            pltpu.VMEM((SW, HHALF), jnp.float32),
            pltpu.VMEM((SW, HHALF), jnp.float32),
            pltpu.VMEM_SHARED((NPAD, HHALF), jnp.float32),
            pltpu.SemaphoreType.DMA,
            pltpu.SemaphoreType.DMA,
        ],
    )


def _prep_body(histT_ref, feats_ref, w2_ref, b2_ref, hn_ref, norm_ref):
    deg = histT_ref[:, 0:1] + histT_ref[:, 1:2] + 1.0
    nrm = lax.rsqrt(jnp.maximum(deg, 1.0))
    row = lax.broadcasted_iota(jnp.int32, (NPAD, 1), 0)
    nrm = jnp.where(row < N_NODES, nrm, 0.0)
    h0 = jnp.dot(feats_ref[...], w2_ref[...],
                 preferred_element_type=jnp.float32) + b2_ref[...]
    hn_ref[...] = h0 * nrm
    norm_ref[...] = nrm


def _prep_call(histT, feats_pad, W2, b2r):
    return pl.pallas_call(
        _prep_body,
        out_shape=[
            jax.ShapeDtypeStruct((NPAD, HIDDEN), jnp.float32),
            jax.ShapeDtypeStruct((NPAD, 1), jnp.float32),
        ],
    )(histT, feats_pad, W2, b2r)


def _main_body(scat0_ref, scat1_ref, hn_ref, norm_ref, wg_ref, bg_ref,
               ct_ref, cb_ref, csq_ref,
               h_ref, q_ref, d2_ref, loss_ref, ind_ref, acc_sm):
    i = pl.program_id(0)
    agg = (scat0_ref[0] + scat1_ref[0] + hn_ref[...]) * norm_ref[...]
    h = jnp.maximum(
        jnp.dot(agg, wg_ref[...], preferred_element_type=jnp.float32)
        + bg_ref[...], 0.0)
    hh = jnp.sum(h * h, axis=1, keepdims=True)
    hc = jnp.dot(h, ct_ref[...], preferred_element_type=jnp.float32)
    d2 = hh - 2.0 * hc + csq_ref[...]
    iota = lax.broadcasted_iota(jnp.int32, (ROWS_D, CODEBOOK), 1)
    mind = jnp.min(d2, axis=1, keepdims=True)
    ind2 = jnp.min(jnp.where(d2 == mind, iota, CODEBOOK),
                   axis=1, keepdims=True)
    oh = (iota == ind2).astype(jnp.float32)
    q = jnp.dot(oh, cb_ref[...], preferred_element_type=jnp.float32,
                precision=lax.Precision.HIGHEST)
    h_ref[...] = h
    q_ref[...] = q
    d2_ref[...] = d2
    ind_ref[...] = ind2

    diff = q - h
    part = jnp.sum(diff * diff)

    @pl.when(i == 0)
    def _():
        acc_sm[0] = 0.0
    acc_sm[0] += part
    loss_ref[...] = jnp.full((1, 1), 1.25 / (N_NODES * HIDDEN),
                             jnp.float32) * acc_sm[0]


def _main_call(scat, hn, nrm, wg, bg2, ct, cb, csq):
    return pl.pallas_call(
        _main_body,
        grid=(GRID_D,),
        in_specs=[
            pl.BlockSpec((1, ROWS_D, HIDDEN), lambda i: (0, i, 0)),
            pl.BlockSpec((1, ROWS_D, HIDDEN), lambda i: (1, i, 0)),
            pl.BlockSpec((ROWS_D, HIDDEN), lambda i: (i, 0)),
            pl.BlockSpec((ROWS_D, 1), lambda i: (i, 0)),
            pl.BlockSpec((HIDDEN, HIDDEN), lambda i: (0, 0)),
            pl.BlockSpec((1, HIDDEN), lambda i: (0, 0)),
            pl.BlockSpec((HIDDEN, CODEBOOK), lambda i: (0, 0)),
            pl.BlockSpec((CODEBOOK, HIDDEN), lambda i: (0, 0)),
            pl.BlockSpec((1, CODEBOOK), lambda i: (0, 0)),
        ],
        out_specs=[
            pl.BlockSpec((ROWS_D, HIDDEN), lambda i: (i, 0)),
            pl.BlockSpec((ROWS_D, HIDDEN), lambda i: (i, 0)),
            pl.BlockSpec((ROWS_D, CODEBOOK), lambda i: (i, 0)),
            pl.BlockSpec((1, 1), lambda i: (0, 0)),
            pl.BlockSpec((ROWS_D, 1), lambda i: (i, 0)),
        ],
        out_shape=[
            jax.ShapeDtypeStruct((N_NODES, HIDDEN), jnp.float32),
            jax.ShapeDtypeStruct((N_NODES, HIDDEN), jnp.float32),
            jax.ShapeDtypeStruct((N_NODES, CODEBOOK), jnp.float32),
            jax.ShapeDtypeStruct((1, 1), jnp.float32),
            jax.ShapeDtypeStruct((N_NODES, 1), jnp.int32),
        ],
        scratch_shapes=[pltpu.SMEM((1,), jnp.float32)],
    )(scat, scat, hn, nrm, wg, bg2, ct, cb, csq)


def kernel(feats, edge_index, epoch, W2, b2, Wg, bg, codebook):
    del epoch
    ei = edge_index.astype(jnp.int32)
    src, dst = ei[0], ei[1]

    endp = ei.reshape(-1)
    ones_w = jnp.ones((HW,), jnp.float32)
    zeros_n = jnp.zeros((NPAD,), jnp.float32)
    hist = _hist_call()(endp, ones_w, zeros_n)

    feats_pad = jnp.pad(feats, ((0, NPAD - N_NODES), (0, 0)))
    hn, nrm = _prep_call(hist.T, feats_pad, W2, b2[None, :])

    half_pad = (PAIRS_PAD - PAIRS) // 2
    ar = jnp.arange(half_pad, dtype=jnp.int32)
    pad_src = N_NODES + (ar % NZERO)
    pad_dst = ar % N_NODES
    srcs_all = jnp.concatenate([src, pad_src, dst, pad_src])
    dsts_all = jnp.concatenate([dst, pad_dst, src, pad_dst])
    zeros_nh = jnp.zeros((NPAD, HHALF), jnp.float32)
    scat_a = _scat_call()(srcs_all, dsts_all, hn[:, :HHALF], zeros_nh)
    scat_b = _scat_call()(srcs_all, dsts_all, hn[:, HHALF:], zeros_nh)
    scat = jnp.concatenate([scat_a, scat_b], axis=2)

    bg2 = bg[None, :]
    ct = codebook.T
    csq = jnp.sum(codebook * codebook, axis=1)[None, :]

    h, q, d2, loss, ind = _main_call(
        scat, hn[:N_NODES], nrm[:N_NODES], Wg, bg2, ct, codebook, csq)
    return h, q, d2, loss[0, 0], ind[:, 0]
